# baseline (speedup 1.0000x reference)
import functools

import jax
import jax.numpy as jnp
from jax import lax
from jax.experimental import pallas as pl
from jax.experimental.pallas import tpu as pltpu
from jax.experimental.pallas import tpu_sc as plsc

MAXLEN = 200
EMB = 64
BATCH_LANES = 128
NUM_TILES = 32
NGBUF = 4


def _sc_gather_add(x_t, token_table, pos_table, batch):
    nb = batch // BATCH_LANES
    assert nb == NUM_TILES
    mesh = plsc.VectorSubcoreMesh(core_axis_name="c", subcore_axis_name="s")

    @functools.partial(
        pl.kernel,
        out_type=jax.ShapeDtypeStruct((MAXLEN, batch, EMB), jnp.float32),
        mesh=mesh,
        compiler_params=pltpu.CompilerParams(use_tc_tiling_on_sc=False,
                                             needs_layout_passes=False),
        scratch_types=[
            pltpu.VMEM((MAXLEN, BATCH_LANES), jnp.int32),
            pltpu.VMEM((MAXLEN, EMB), jnp.float32),
        ] + [pltpu.VMEM((BATCH_LANES, EMB), jnp.float32) for _ in range(NGBUF)]
          + [pltpu.SemaphoreType.DMA for _ in range(2 * NGBUF)],
    )
    def k(x_hbm, tok_hbm, pos_hbm, out_hbm, idx_v, pos_v, *bufs_and_sems):
        gbufs = bufs_and_sems[:NGBUF]
        gsems = bufs_and_sems[NGBUF:2 * NGBUF]
        osems = bufs_and_sems[2 * NGBUF:]
        wid = lax.axis_index("s") * 2 + lax.axis_index("c")
        pltpu.sync_copy(x_hbm.at[:, pl.ds(wid * BATCH_LANES, BATCH_LANES)],
                        idx_v)
        pltpu.sync_copy(pos_hbm, pos_v)

        def issue_gather(p, b):
            pltpu.async_copy(tok_hbm.at[idx_v.at[p]], gbufs[b], gsems[b])

        def wait_gather(p, b):
            pltpu.make_async_copy(tok_hbm.at[idx_v.at[p]], gbufs[b],
                                  gsems[b]).wait()

        def issue_out(p, b):
            pltpu.async_copy(
                gbufs[b], out_hbm.at[p, pl.ds(wid * BATCH_LANES, BATCH_LANES)],
                osems[b])

        def wait_out(p, b):
            pltpu.make_async_copy(
                gbufs[b], out_hbm.at[p, pl.ds(wid * BATCH_LANES, BATCH_LANES)],
                osems[b]).wait()

        issue_gather(0, 0)
        issue_gather(1, 1)

        @pl.loop(0, MAXLEN, step=NGBUF)
        def _grp(g):
            for b in range(NGBUF):
                p = g + b
                bp = (b + 2) % NGBUF
                wait_gather(p, b)

                @pl.when(p + 2 < MAXLEN)
                def _prefetch():
                    @pl.when(p >= 2)
                    def _drain():
                        wait_out(p - 2, bp)
                    issue_gather(p + 2, bp)

                pos_es = [pos_v[p, pl.ds(16 * c, 16)] for c in range(EMB // 16)]

                @pl.loop(0, BATCH_LANES, unroll=8)
                def _row(r):
                    for c in range(EMB // 16):
                        plsc.addupdate(gbufs[b].at[r, pl.ds(16 * c, 16)],
                                       pos_es[c])

                issue_out(p, b)

        for b in range(NGBUF):
            wait_out(MAXLEN - NGBUF + b, b)

    return k(x_t, token_table, pos_table)


def _tc_transpose(inter, batch):
    nb = batch // BATCH_LANES

    def body(in_ref, out_ref):
        for w in range(nb):
            blk = in_ref[0, pl.ds(w * BATCH_LANES, BATCH_LANES), :]
            out_ref[0, :, w] = jnp.transpose(blk, (1, 0)).reshape(
                EMB // 8, 8, BATCH_LANES)

    return pl.pallas_call(
        body,
        grid=(MAXLEN,),
        in_specs=[pl.BlockSpec((1, batch, EMB), lambda p: (p, 0, 0))],
        out_specs=pl.BlockSpec((1, EMB // 8, nb, 8, BATCH_LANES),
                               lambda p: (p, 0, 0, 0, 0)),
        out_shape=jax.ShapeDtypeStruct((MAXLEN, EMB // 8, nb, 8, BATCH_LANES),
                                       jnp.float32),
    )(inter)


def kernel(x, token_table, pos_table):
    batch, seq = x.shape
    if seq < MAXLEN:
        x = jnp.pad(x, ((0, 0), (0, MAXLEN - seq)))
    else:
        x = x[:, :MAXLEN]
    x_t = x.T.astype(jnp.int32)
    inter = _sc_gather_add(x_t, token_table, pos_table, batch)
    out5 = _tc_transpose(inter, batch)
    return out5.transpose(2, 4, 0, 1, 3).reshape(batch, MAXLEN, EMB)


# speedup vs baseline: 1.3287x; 1.3287x over previous
import functools

import jax
import jax.numpy as jnp
from jax import lax
from jax.experimental import pallas as pl
from jax.experimental.pallas import tpu as pltpu
from jax.experimental.pallas import tpu_sc as plsc

MAXLEN = 200
EMB = 64
LANES = 128
NUM_TILES = 32
NBUF = 4
SPLIT = (104, 96)


def _tok_pos_embed(x_flat, tok_padded, pos_table):
    total = x_flat.shape[0]
    rows_per_tile = total // NUM_TILES
    nchunk = 2 * (rows_per_tile // MAXLEN)
    mesh = plsc.VectorSubcoreMesh(core_axis_name="c", subcore_axis_name="s")

    @functools.partial(
        pl.kernel,
        out_type=jax.ShapeDtypeStruct((total, LANES), jnp.float32),
        mesh=mesh,
        scratch_types=[
            pltpu.VMEM((rows_per_tile,), jnp.int32),
            pltpu.VMEM((MAXLEN, EMB), jnp.float32),
        ] + [pltpu.VMEM((SPLIT[0], LANES), jnp.float32) for _ in range(NBUF)]
          + [pltpu.SemaphoreType.DMA for _ in range(2 * NBUF)],
    )
    def k(x_hbm, tok_hbm, pos_hbm, out_hbm, idx_v, pos_v, *bufs_and_sems):
        bufs = bufs_and_sems[:NBUF]
        gsems = bufs_and_sems[NBUF:2 * NBUF]
        osems = bufs_and_sems[2 * NBUF:]
        wid = lax.axis_index("s") * 2 + lax.axis_index("c")
        base = wid * rows_per_tile
        pltpu.sync_copy(x_hbm.at[pl.ds(base, rows_per_tile)], idx_v)
        pltpu.sync_copy(pos_hbm, pos_v)

        def chunk_off(c):
            return (c // 2) * MAXLEN + (c % 2) * SPLIT[0]

        def issue_gather(c, b, n):
            pltpu.async_copy(
                tok_hbm.at[idx_v.at[pl.ds(chunk_off(c), n)]],
                bufs[b].at[pl.ds(0, n)], gsems[b])

        def wait_gather(c, b, n):
            pltpu.make_async_copy(
                tok_hbm.at[idx_v.at[pl.ds(chunk_off(c), n)]],
                bufs[b].at[pl.ds(0, n)], gsems[b]).wait()

        def issue_out(c, b, n):
            pltpu.async_copy(
                bufs[b].at[pl.ds(0, n)],
                out_hbm.at[pl.ds(base + chunk_off(c), n)], osems[b])

        def wait_out(c, b, n):
            pltpu.make_async_copy(
                bufs[b].at[pl.ds(0, n)],
                out_hbm.at[pl.ds(base + chunk_off(c), n)], osems[b]).wait()

        issue_gather(0, 0, SPLIT[0])
        issue_gather(1, 1, SPLIT[1])

        @pl.loop(0, nchunk, step=NBUF)
        def _grp(g):
            for b in range(NBUF):
                c = g + b
                n = SPLIT[b % 2]
                p0 = (b % 2) * SPLIT[0]
                bp = (b + 2) % NBUF
                np_ = SPLIT[bp % 2]
                wait_gather(c, b, n)

                @pl.when(c + 2 < nchunk)
                def _prefetch():
                    @pl.when(c >= 2)
                    def _drain():
                        wait_out(c - 2, bp, np_)
                    issue_gather(c + 2, bp, np_)

                @pl.loop(0, n, unroll=4)
                def _row(r):
                    for col in range(0, EMB, 16):
                        plsc.addupdate(bufs[b].at[r, pl.ds(col, 16)],
                                       pos_v[p0 + r, pl.ds(col, 16)])

                issue_out(c, b, n)

        for b in range(NBUF):
            wait_out(nchunk - NBUF + b, b, SPLIT[b % 2])

    return k(x_flat, tok_padded, pos_table)


def kernel(x, token_table, pos_table):
    batch, seq = x.shape
    if seq < MAXLEN:
        x = jnp.pad(x, ((0, 0), (0, MAXLEN - seq)))
    else:
        x = x[:, :MAXLEN]
    x_flat = x.reshape(-1).astype(jnp.int32)
    tok_padded = jnp.pad(token_table, ((0, 0), (0, LANES - EMB)))
    out = _tok_pos_embed(x_flat, tok_padded, pos_table)
    return out[:, :EMB].reshape(batch, MAXLEN, EMB)
